# speedup vs baseline: 1.5720x; 1.5720x over previous
import jax
import jax.numpy as jnp
from jax import lax
from jax.experimental import pallas as pl
from jax.experimental.pallas import tpu as pltpu
from jax.experimental.pallas import tpu_sc as plsc

B, S = 4, 4096
D_MODEL = 1024
N_TOK = B * S

_info = plsc.get_sparse_core_info()
NC, NS = _info.num_cores, _info.num_subcores
NW = NC * NS
ROWS_PER_W = N_TOK // NW
CHUNK = 32
N_CHUNKS = ROWS_PER_W // CHUNK


def _emb_kernel(table_hbm, idx_hbm, out_hbm, idx_v, buf0, buf1,
                gsem0, gsem1, osem0, osem1):
    wid = lax.axis_index("s") * NC + lax.axis_index("c")
    pltpu.sync_copy(idx_hbm.at[wid], idx_v)

    bufs = (buf0, buf1)
    gsems = (gsem0, gsem1)
    osems = (osem0, osem1)
    base = wid * ROWS_PER_W

    gathers = [None, None]
    outs = [None, None]

    gathers[0] = pltpu.async_copy(table_hbm.at[idx_v.at[0]], bufs[0], gsems[0])
    for g in range(N_CHUNKS):
        b = g & 1
        nb = (g + 1) & 1
        gathers[b].wait()
        if g >= 1:
            outs[nb].wait()
        if g + 1 < N_CHUNKS:
            gathers[nb] = pltpu.async_copy(
                table_hbm.at[idx_v.at[g + 1]], bufs[nb], gsems[nb])
        outs[b] = pltpu.async_copy(
            bufs[b], out_hbm.at[pl.ds(base + g * CHUNK, CHUNK)], osems[b])
    outs[(N_CHUNKS - 1) & 1].wait()


def kernel(tokens, W_E):
    tok = tokens.reshape(NW, N_CHUNKS, CHUNK).astype(jnp.int32)
    mesh = plsc.VectorSubcoreMesh(core_axis_name="c", subcore_axis_name="s")
    out = pl.kernel(
        _emb_kernel,
        mesh=mesh,
        out_type=jax.ShapeDtypeStruct((N_TOK, D_MODEL), jnp.float32),
        scratch_types=[
            pltpu.VMEM((N_CHUNKS, CHUNK), jnp.int32),
            pltpu.VMEM((CHUNK, D_MODEL), jnp.float32),
            pltpu.VMEM((CHUNK, D_MODEL), jnp.float32),
            pltpu.SemaphoreType.DMA,
            pltpu.SemaphoreType.DMA,
            pltpu.SemaphoreType.DMA,
            pltpu.SemaphoreType.DMA,
        ],
    )(W_E, tok)
    return out.reshape(B, S, D_MODEL)


# speedup vs baseline: 1.6184x; 1.0295x over previous
import jax
import jax.numpy as jnp
from jax import lax
from jax.experimental import pallas as pl
from jax.experimental.pallas import tpu as pltpu
from jax.experimental.pallas import tpu_sc as plsc

B, S = 4, 4096
D_MODEL = 1024
N_TOK = B * S

_info = plsc.get_sparse_core_info()
NC, NS = _info.num_cores, _info.num_subcores
NW = NC * NS
ROWS_PER_W = N_TOK // NW
CHUNK = 32
N_CHUNKS = ROWS_PER_W // CHUNK


NBUF = 3


def _emb_kernel(table_hbm, idx_hbm, out_hbm, idx_v, buf0, buf1, buf2,
                gsem0, gsem1, gsem2, osem0, osem1, osem2):
    wid = lax.axis_index("s") * NC + lax.axis_index("c")
    pltpu.sync_copy(idx_hbm.at[wid], idx_v)

    bufs = (buf0, buf1, buf2)
    gsems = (gsem0, gsem1, gsem2)
    osems = (osem0, osem1, osem2)
    base = wid * ROWS_PER_W

    def gather(g):
        b = g % NBUF
        return pltpu.async_copy(table_hbm.at[idx_v.at[g]], bufs[b], gsems[b])

    gathers = [None] * N_CHUNKS
    outs = [None] * N_CHUNKS
    gathers[0] = gather(0)
    gathers[1] = gather(1)
    for g in range(N_CHUNKS):
        b = g % NBUF
        gathers[g].wait()
        outs[g] = pltpu.async_copy(
            bufs[b], out_hbm.at[pl.ds(base + g * CHUNK, CHUNK)], osems[b])
        if g + 2 < N_CHUNKS:
            if g >= 1:
                outs[g - 1].wait()
            gathers[g + 2] = gather(g + 2)
    outs[N_CHUNKS - 2].wait()
    outs[N_CHUNKS - 1].wait()


def kernel(tokens, W_E):
    tok = tokens.reshape(NW, N_CHUNKS, CHUNK).astype(jnp.int32)
    mesh = plsc.VectorSubcoreMesh(core_axis_name="c", subcore_axis_name="s")
    out = pl.kernel(
        _emb_kernel,
        mesh=mesh,
        out_type=jax.ShapeDtypeStruct((N_TOK, D_MODEL), jnp.float32),
        scratch_types=[
            pltpu.VMEM((N_CHUNKS, CHUNK), jnp.int32),
            pltpu.VMEM((CHUNK, D_MODEL), jnp.float32),
            pltpu.VMEM((CHUNK, D_MODEL), jnp.float32),
            pltpu.VMEM((CHUNK, D_MODEL), jnp.float32),
            pltpu.SemaphoreType.DMA,
            pltpu.SemaphoreType.DMA,
            pltpu.SemaphoreType.DMA,
            pltpu.SemaphoreType.DMA,
            pltpu.SemaphoreType.DMA,
            pltpu.SemaphoreType.DMA,
        ],
    )(W_E, tok)
    return out.reshape(B, S, D_MODEL)


# speedup vs baseline: 1.6800x; 1.0381x over previous
import jax
import jax.numpy as jnp
from jax import lax
from jax.experimental import pallas as pl
from jax.experimental.pallas import tpu as pltpu
from jax.experimental.pallas import tpu_sc as plsc

B, S = 4, 4096
D_MODEL = 1024
N_TOK = B * S

_info = plsc.get_sparse_core_info()
NC, NS = _info.num_cores, _info.num_subcores
NW = NC * NS
ROWS_PER_W = N_TOK // NW
W_PER_ROW = S // ROWS_PER_W
CHUNK = 16
N_CHUNKS = ROWS_PER_W // CHUNK
NBUF = 6


def _emb_kernel(table_hbm, idx_hbm, out_hbm, idx_v, *rest):
    bufs = rest[:NBUF]
    gsems = rest[NBUF:2 * NBUF]
    osems = rest[2 * NBUF:3 * NBUF]
    wid = lax.axis_index("s") * NC + lax.axis_index("c")
    pltpu.sync_copy(
        idx_hbm.at[wid // W_PER_ROW,
                   pl.ds((wid % W_PER_ROW) * ROWS_PER_W, ROWS_PER_W)],
        idx_v)
    base = wid * ROWS_PER_W

    def gather(g):
        b = g % NBUF
        return pltpu.async_copy(
            table_hbm.at[idx_v.at[pl.ds(g * CHUNK, CHUNK)]], bufs[b], gsems[b])

    gathers = [None] * N_CHUNKS
    outs = [None] * N_CHUNKS
    for h in range(min(NBUF - 1, N_CHUNKS)):
        gathers[h] = gather(h)
    for g in range(N_CHUNKS):
        b = g % NBUF
        gathers[g].wait()
        outs[g] = pltpu.async_copy(
            bufs[b], out_hbm.at[pl.ds(base + g * CHUNK, CHUNK)], osems[b])
        h = g + NBUF - 1
        if h < N_CHUNKS:
            if h >= NBUF:
                outs[h - NBUF].wait()
            gathers[h] = gather(h)
    for g in range(max(0, N_CHUNKS - NBUF), N_CHUNKS):
        if outs[g] is not None:
            outs[g].wait()


def kernel(tokens, W_E):
    mesh = plsc.VectorSubcoreMesh(core_axis_name="c", subcore_axis_name="s")
    scratch = (
        [pltpu.VMEM((ROWS_PER_W,), jnp.int32)]
        + [pltpu.VMEM((CHUNK, D_MODEL), jnp.float32) for _ in range(NBUF)]
        + [pltpu.SemaphoreType.DMA for _ in range(2 * NBUF)]
    )
    out = pl.kernel(
        _emb_kernel,
        mesh=mesh,
        out_type=jax.ShapeDtypeStruct((N_TOK, D_MODEL), jnp.float32),
        scratch_types=scratch,
    )(W_E, tokens)
    return out.reshape(B, S, D_MODEL)
